# baseline (speedup 1.0000x reference)
import functools

import jax
import jax.numpy as jnp
from jax import lax
from jax.experimental import pallas as pl
from jax.experimental.pallas import tpu as pltpu
from jax.experimental.pallas import tpu_sc as plsc

SEQ_LEN = 4096
BATCH = 2
D_MODEL = 1024
R = 8
NW = 32
ROWS_PER_W = SEQ_LEN // NW
N_CHUNKS = ROWS_PER_W // R
LANES = 16
KVECS = D_MODEL // LANES

_mesh = plsc.VectorSubcoreMesh(core_axis_name="c", subcore_axis_name="s")


@functools.partial(
    pl.kernel,
    mesh=_mesh,
    out_type=jax.ShapeDtypeStruct((SEQ_LEN, BATCH, D_MODEL), jnp.float32),
    scratch_types=[
        pltpu.VMEM((R, BATCH, D_MODEL), jnp.float32),
        pltpu.VMEM((R, D_MODEL), jnp.float32),
    ],
)
def _sc_add(x_hbm, pos_hbm, out_hbm, xbuf, posbuf):
    wid = lax.axis_index("c") * 16 + lax.axis_index("s")
    base = wid * ROWS_PER_W

    def chunk_body(c, carry):
        s0 = base + c * R
        pltpu.sync_copy(x_hbm.at[pl.ds(s0, R)], xbuf)
        pltpu.sync_copy(pos_hbm.at[pl.ds(s0, R)], posbuf)
        for r in range(R):
            for b in range(BATCH):
                def k_body(k, inner):
                    off = k * LANES
                    xbuf[r, b, pl.ds(off, LANES)] = (
                        xbuf[r, b, pl.ds(off, LANES)]
                        + posbuf[r, pl.ds(off, LANES)]
                    )
                    return inner

                lax.fori_loop(0, KVECS, k_body, 0, unroll=8)
        pltpu.sync_copy(xbuf, out_hbm.at[pl.ds(s0, R)])
        return carry

    lax.fori_loop(0, N_CHUNKS, chunk_body, 0)


def kernel(x, pos_embedding):
    return _sc_add(x, pos_embedding)


# speedup vs baseline: 5.1082x; 5.1082x over previous
import jax
import jax.numpy as jnp
from jax.experimental import pallas as pl
from jax.experimental.pallas import tpu as pltpu

BLOCK_S = 2048
BLOCK_D = 512


def _add_kernel(x_ref, pos_ref, out_ref):
    pos = pos_ref[...]
    out_ref[:, 0, :] = x_ref[:, 0, :] + pos
    out_ref[:, 1, :] = x_ref[:, 1, :] + pos


def kernel(x, pos_embedding):
    seq_len, batch, d_model = x.shape
    grid = (seq_len // BLOCK_S, d_model // BLOCK_D)
    return pl.pallas_call(
        _add_kernel,
        grid=grid,
        in_specs=[
            pl.BlockSpec((BLOCK_S, batch, BLOCK_D), lambda i, j: (i, 0, j)),
            pl.BlockSpec((BLOCK_S, BLOCK_D), lambda i, j: (i, j)),
        ],
        out_specs=pl.BlockSpec((BLOCK_S, batch, BLOCK_D), lambda i, j: (i, 0, j)),
        out_shape=jax.ShapeDtypeStruct((seq_len, batch, d_model), x.dtype),
        compiler_params=pltpu.CompilerParams(
            dimension_semantics=("arbitrary", "arbitrary"),
        ),
    )(x, pos_embedding)


# speedup vs baseline: 5.3202x; 1.0415x over previous
import jax
import jax.numpy as jnp
from jax.experimental import pallas as pl
from jax.experimental.pallas import tpu as pltpu

BLOCK_S = 1024


def _add_kernel(x_ref, pos_ref, out_ref):
    pos = pos_ref[...]
    out_ref[:, 0, :] = x_ref[:, 0, :] + pos
    out_ref[:, 1, :] = x_ref[:, 1, :] + pos


def kernel(x, pos_embedding):
    seq_len, batch, d_model = x.shape
    grid = (seq_len // BLOCK_S,)
    return pl.pallas_call(
        _add_kernel,
        grid=grid,
        in_specs=[
            pl.BlockSpec((BLOCK_S, batch, d_model), lambda i: (i, 0, 0)),
            pl.BlockSpec((BLOCK_S, d_model), lambda i: (i, 0)),
        ],
        out_specs=pl.BlockSpec((BLOCK_S, batch, d_model), lambda i: (i, 0, 0)),
        out_shape=jax.ShapeDtypeStruct((seq_len, batch, d_model), x.dtype),
        compiler_params=pltpu.CompilerParams(
            dimension_semantics=("arbitrary",),
        ),
    )(x, pos_embedding)
